# Initial kernel scaffold; baseline (speedup 1.0000x reference)
import functools

import jax
import jax.numpy as jnp
from jax import lax
from jax.experimental import pallas as pl
from jax.experimental.pallas import tpu as pltpu
from jax.experimental.pallas import tpu_sc as plsc

N = 10000
E = 320000
D_FEAT = 128
D_EDGE = 16

NC = 2
NS = 16
NW = NC * NS
EPT = E // NW
CHUNK = 128
NFULL = EPT // CHUNK
TAIL = EPT - NFULL * CHUNK
N_PAD = 10016
ROWS_PER_SUB = N_PAD // NS

_sc_mesh = plsc.VectorSubcoreMesh(core_axis_name="c", subcore_axis_name="s")


@functools.partial(
    pl.kernel,
    out_type=jax.ShapeDtypeStruct((NC, N_PAD, D_EDGE), jnp.float32),
    mesh=_sc_mesh,
    scratch_types=[
        pltpu.VMEM_SHARED((N_PAD, D_EDGE), jnp.float32),
        pltpu.VMEM((ROWS_PER_SUB, D_EDGE), jnp.float32),
        pltpu.VMEM((CHUNK, D_EDGE), jnp.float32),
        pltpu.VMEM((CHUNK,), jnp.int32),
        pltpu.VMEM((TAIL, D_EDGE), jnp.float32),
        pltpu.VMEM((TAIL,), jnp.int32),
    ],
)
def _sc_segment_sum(ea_hbm, recv_hbm, out_hbm, acc, stage, rows_v, idx_v,
                    rows_t, idx_t):
    c = lax.axis_index("c")
    s = lax.axis_index("s")
    wid = s * NC + c

    def _zero_row(i, _):
        stage[i, :] = jnp.zeros((D_EDGE,), jnp.float32)
        return 0

    lax.fori_loop(0, ROWS_PER_SUB, _zero_row, 0)
    pltpu.sync_copy(stage, acc.at[pl.ds(s * ROWS_PER_SUB, ROWS_PER_SUB)])
    plsc.subcore_barrier()

    base = wid * EPT

    def _chunk(k, _):
        off = base + k * CHUNK
        pltpu.sync_copy(recv_hbm.at[pl.ds(off, CHUNK)], idx_v)
        pltpu.sync_copy(ea_hbm.at[pl.ds(off, CHUNK)], rows_v)
        pltpu.sync_copy(rows_v, acc.at[idx_v], add=True)
        return 0

    lax.fori_loop(0, NFULL, _chunk, 0)

    off = base + NFULL * CHUNK
    pltpu.sync_copy(recv_hbm.at[pl.ds(off, TAIL)], idx_t)
    pltpu.sync_copy(ea_hbm.at[pl.ds(off, TAIL)], rows_t)
    pltpu.sync_copy(rows_t, acc.at[idx_t], add=True)

    plsc.subcore_barrier()

    row0 = s * ROWS_PER_SUB
    pltpu.sync_copy(acc.at[pl.ds(row0, ROWS_PER_SUB)], stage)
    pltpu.sync_copy(stage, out_hbm.at[c, pl.ds(row0, ROWS_PER_SUB)])


def _mlp_body(x_ref, p0_ref, p1_ref, wx_ref, we_ref, b_ref, o_ref):
    agg = p0_ref[...] + p1_ref[...]
    o_ref[...] = (
        jnp.dot(x_ref[...], wx_ref[...], preferred_element_type=jnp.float32)
        + jnp.dot(agg, we_ref[...], preferred_element_type=jnp.float32)
        + b_ref[...]
    )


@jax.jit
def _tc_mlp(x, p0, p1, wx, we, b2d):
    return pl.pallas_call(
        _mlp_body,
        out_shape=jax.ShapeDtypeStruct((N, D_FEAT), jnp.float32),
    )(x, p0, p1, wx, we, b2d)


@jax.jit
def kernel(x, edge_index, edge_attr, pos, W, b):
    recv = edge_index[1]
    partials = _sc_segment_sum(edge_attr, recv)
    p0 = partials[0, :N]
    p1 = partials[1, :N]
    x_ = _tc_mlp(x, p0, p1, W[:D_FEAT], W[D_FEAT:], b[None, :])
    return (x_, edge_attr, edge_index, pos)


# speedup vs baseline: 3.2960x; 3.2960x over previous
import functools

import jax
import jax.numpy as jnp
from jax import lax
from jax.experimental import pallas as pl
from jax.experimental.pallas import tpu as pltpu
from jax.experimental.pallas import tpu_sc as plsc

N = 10000
E = 320000
D_FEAT = 128
D_EDGE = 16

NC = 2
NS = 16
NW = NC * NS
EPT = E // NW
CHUNK = 128
NFULL = EPT // CHUNK
TAIL = EPT - NFULL * CHUNK
N_PAD = 10112
ROWS_PER_SUB = N_PAD // NS

_sc_mesh = plsc.VectorSubcoreMesh(core_axis_name="c", subcore_axis_name="s")


@functools.partial(
    pl.kernel,
    out_type=jax.ShapeDtypeStruct((NC, N_PAD, D_EDGE), jnp.float32),
    mesh=_sc_mesh,
    scratch_types=[
        pltpu.VMEM_SHARED((N_PAD, D_EDGE), jnp.float32),
        pltpu.VMEM((ROWS_PER_SUB, D_EDGE), jnp.float32),
        pltpu.VMEM((CHUNK, D_EDGE), jnp.float32),
        pltpu.VMEM((CHUNK,), jnp.int32),
        pltpu.VMEM((TAIL, D_EDGE), jnp.float32),
        pltpu.VMEM((TAIL,), jnp.int32),
    ],
)
def _sc_segment_sum(ea_hbm, recv_hbm, out_hbm, acc, stage, rows_v, idx_v,
                    rows_t, idx_t):
    c = lax.axis_index("c")
    s = lax.axis_index("s")
    wid = s * NC + c

    def _zero_row(i, _):
        stage[i, :] = jnp.zeros((D_EDGE,), jnp.float32)
        return 0

    lax.fori_loop(0, ROWS_PER_SUB, _zero_row, 0)
    pltpu.sync_copy(stage, acc.at[pl.ds(s * ROWS_PER_SUB, ROWS_PER_SUB)])
    plsc.subcore_barrier()

    base = wid * EPT

    def _chunk(k, _):
        off = base + k * CHUNK
        pltpu.sync_copy(recv_hbm.at[pl.ds(off, CHUNK)], idx_v)
        pltpu.sync_copy(ea_hbm.at[pl.ds(off, CHUNK)], rows_v)
        pltpu.sync_copy(rows_v, acc.at[idx_v], add=True)
        return 0

    lax.fori_loop(0, NFULL, _chunk, 0)

    off = base + NFULL * CHUNK
    pltpu.sync_copy(recv_hbm.at[pl.ds(off, TAIL)], idx_t)
    pltpu.sync_copy(ea_hbm.at[pl.ds(off, TAIL)], rows_t)
    pltpu.sync_copy(rows_t, acc.at[idx_t], add=True)

    plsc.subcore_barrier()

    row0 = s * ROWS_PER_SUB
    pltpu.sync_copy(acc.at[pl.ds(row0, ROWS_PER_SUB)], stage)
    pltpu.sync_copy(stage, out_hbm.at[c, pl.ds(row0, ROWS_PER_SUB)])


def _mlp_body(x_ref, p0_ref, p1_ref, wx_ref, we_ref, b_ref, o_ref):
    agg = p0_ref[...] + p1_ref[...]
    o_ref[...] = (
        jnp.dot(x_ref[...], wx_ref[...], preferred_element_type=jnp.float32)
        + jnp.dot(agg, we_ref[...], preferred_element_type=jnp.float32)
        + b_ref[...]
    )


@jax.jit
def _tc_mlp(x, p0, p1, wx, we, b2d):
    return pl.pallas_call(
        _mlp_body,
        out_shape=jax.ShapeDtypeStruct((N, D_FEAT), jnp.float32),
    )(x, p0, p1, wx, we, b2d)


@jax.jit
def kernel(x, edge_index, edge_attr, pos, W, b):
    recv = edge_index[1]
    partials = _sc_segment_sum(edge_attr, recv)
    p0 = partials[0, :N]
    p1 = partials[1, :N]
    x_ = _tc_mlp(x, p0, p1, W[:D_FEAT], W[D_FEAT:], b[None, :])
    return (x_, edge_attr, edge_index, pos)


# speedup vs baseline: 3.5508x; 1.0773x over previous
import functools

import jax
import jax.numpy as jnp
from jax import lax
from jax.experimental import pallas as pl
from jax.experimental.pallas import tpu as pltpu
from jax.experimental.pallas import tpu_sc as plsc

N = 10000
E = 320000
D_FEAT = 128
D_EDGE = 16

NC = 2
NS = 16
NW = NC * NS
CHUNK = 128
NCHUNKS = E // CHUNK
CPT = NCHUNKS // NW
XTRA = NCHUNKS - CPT * NW
N_PAD = 10112
ROWS_PER_SUB = N_PAD // NS

_sc_mesh = plsc.VectorSubcoreMesh(core_axis_name="c", subcore_axis_name="s")


@functools.partial(
    pl.kernel,
    out_type=jax.ShapeDtypeStruct((NC, N_PAD, D_EDGE), jnp.float32),
    mesh=_sc_mesh,
    compiler_params=pltpu.CompilerParams(use_tc_tiling_on_sc=False),
    scratch_types=[
        pltpu.VMEM_SHARED((N_PAD, D_EDGE), jnp.float32),
        pltpu.VMEM((CHUNK, D_EDGE), jnp.float32),
        pltpu.VMEM((CHUNK,), jnp.int32),
    ],
)
def _sc_segment_sum(z_hbm, ea_hbm, recv_hbm, out_hbm, acc, rows_v, idx_v):
    c = lax.axis_index("c")
    s = lax.axis_index("s")
    wid = s * NC + c
    base = wid * CPT * CHUNK
    row0 = s * ROWS_PER_SUB

    pltpu.sync_copy(z_hbm.at[pl.ds(row0, ROWS_PER_SUB)],
                    acc.at[pl.ds(row0, ROWS_PER_SUB)])
    plsc.subcore_barrier()

    def _chunk(k, _):
        off = jnp.where(k < CPT, base + k * CHUNK, (NW * CPT + wid) * CHUNK)
        pltpu.sync_copy(recv_hbm.at[pl.ds(off, CHUNK)], idx_v)
        pltpu.sync_copy(ea_hbm.at[pl.ds(off, CHUNK)], rows_v)
        pltpu.sync_copy(rows_v, acc.at[idx_v], add=True)
        return 0

    nchunks_this = CPT + jnp.where(wid < XTRA, 1, 0)
    lax.fori_loop(0, nchunks_this, _chunk, 0)

    plsc.subcore_barrier()

    pltpu.sync_copy(acc.at[pl.ds(row0, ROWS_PER_SUB)],
                    out_hbm.at[c, pl.ds(row0, ROWS_PER_SUB)])


def _mlp_body(x_ref, p0_ref, p1_ref, wx_ref, we_ref, b_ref, o_ref):
    agg = p0_ref[...] + p1_ref[...]
    o_ref[...] = (
        jnp.dot(x_ref[...], wx_ref[...], preferred_element_type=jnp.float32)
        + jnp.dot(agg, we_ref[...], preferred_element_type=jnp.float32)
        + b_ref[...]
    )


@jax.jit
def _tc_mlp(x, p0, p1, wx, we, b2d):
    return pl.pallas_call(
        _mlp_body,
        out_shape=jax.ShapeDtypeStruct((N, D_FEAT), jnp.float32),
    )(x, p0, p1, wx, we, b2d)


@jax.jit
def kernel(x, edge_index, edge_attr, pos, W, b):
    recv = edge_index[1]
    zrows = jnp.zeros((N_PAD, D_EDGE), jnp.float32)
    partials = _sc_segment_sum(zrows, edge_attr, recv)
    p0 = partials[0, :N]
    p1 = partials[1, :N]
    x_ = _tc_mlp(x, p0, p1, W[:D_FEAT], W[D_FEAT:], b[None, :])
    return (x_, edge_attr, edge_index, pos)


# speedup vs baseline: 4.6098x; 1.2983x over previous
import functools

import jax
import jax.numpy as jnp
from jax import lax
from jax.experimental import pallas as pl
from jax.experimental.pallas import tpu as pltpu
from jax.experimental.pallas import tpu_sc as plsc

N = 10000
E = 320000
D_FEAT = 128
D_EDGE = 16

NC = 2
NS = 16
NW = NC * NS
CHUNK = 128
NCHUNKS = E // CHUNK
CPT = NCHUNKS // NW
XTRA = NCHUNKS - CPT * NW
N_PAD = 10112
ROWS_PER_SUB = N_PAD // NS

_sc_mesh = plsc.VectorSubcoreMesh(core_axis_name="c", subcore_axis_name="s")


@functools.partial(
    pl.kernel,
    out_type=jax.ShapeDtypeStruct((NC, N_PAD, D_EDGE), jnp.float32),
    mesh=_sc_mesh,
    compiler_params=pltpu.CompilerParams(use_tc_tiling_on_sc=False),
    scratch_types=[
        pltpu.VMEM_SHARED((N_PAD, D_EDGE), jnp.float32),
        pltpu.VMEM((CHUNK, D_EDGE), jnp.float32),
        pltpu.VMEM((CHUNK, D_EDGE), jnp.float32),
        pltpu.VMEM((CHUNK,), jnp.int32),
        pltpu.VMEM((CHUNK,), jnp.int32),
        pltpu.SemaphoreType.DMA,
        pltpu.SemaphoreType.DMA,
    ],
)
def _sc_segment_sum(z_hbm, ea_hbm, recv_hbm, out_hbm, acc, rows_a, rows_b,
                    idx_a, idx_b, sem_a, sem_b):
    c = lax.axis_index("c")
    s = lax.axis_index("s")
    wid = s * NC + c
    base = wid * CPT * CHUNK
    row0 = s * ROWS_PER_SUB

    row_bufs = (rows_a, rows_b)
    idx_bufs = (idx_a, idx_b)
    sems = (sem_a, sem_b)

    def start_load(k, buf):
        off = base + k * CHUNK
        pltpu.async_copy(recv_hbm.at[pl.ds(off, CHUNK)], idx_bufs[buf],
                         sems[buf])
        pltpu.async_copy(ea_hbm.at[pl.ds(off, CHUNK)], row_bufs[buf],
                         sems[buf])

    def wait_load(buf):
        pltpu.make_async_copy(recv_hbm.at[pl.ds(0, CHUNK)], idx_bufs[buf],
                              sems[buf]).wait()
        pltpu.make_async_copy(ea_hbm.at[pl.ds(0, CHUNK)], row_bufs[buf],
                              sems[buf]).wait()

    start_load(0, 0)
    pltpu.sync_copy(z_hbm.at[pl.ds(row0, ROWS_PER_SUB)],
                    acc.at[pl.ds(row0, ROWS_PER_SUB)])
    plsc.subcore_barrier()

    def _pair(i, _):
        g = 2 * i
        start_load(g + 1, 1)
        wait_load(0)
        pltpu.sync_copy(rows_a, acc.at[idx_a], add=True)
        start_load(jnp.minimum(g + 2, CPT - 2), 0)
        wait_load(1)
        pltpu.sync_copy(rows_b, acc.at[idx_b], add=True)
        return 0

    lax.fori_loop(0, CPT // 2, _pair, 0)
    wait_load(0)

    def _extra(_, __):
        off = (NW * CPT + wid) * CHUNK
        pltpu.sync_copy(recv_hbm.at[pl.ds(off, CHUNK)], idx_a)
        pltpu.sync_copy(ea_hbm.at[pl.ds(off, CHUNK)], rows_a)
        pltpu.sync_copy(rows_a, acc.at[idx_a], add=True)
        return 0

    lax.fori_loop(0, jnp.where(wid < XTRA, 1, 0), _extra, 0)

    plsc.subcore_barrier()

    pltpu.sync_copy(acc.at[pl.ds(row0, ROWS_PER_SUB)],
                    out_hbm.at[c, pl.ds(row0, ROWS_PER_SUB)])


def _mlp_body(x_ref, p0_ref, p1_ref, wx_ref, we_ref, b_ref, o_ref):
    agg = p0_ref[...] + p1_ref[...]
    o_ref[...] = (
        jnp.dot(x_ref[...], wx_ref[...], preferred_element_type=jnp.float32)
        + jnp.dot(agg, we_ref[...], preferred_element_type=jnp.float32)
        + b_ref[...]
    )


@jax.jit
def _tc_mlp(x, p0, p1, wx, we, b2d):
    return pl.pallas_call(
        _mlp_body,
        out_shape=jax.ShapeDtypeStruct((N, D_FEAT), jnp.float32),
    )(x, p0, p1, wx, we, b2d)


@jax.jit
def kernel(x, edge_index, edge_attr, pos, W, b):
    recv = edge_index[1]
    zrows = jnp.zeros((N_PAD, D_EDGE), jnp.float32)
    partials = _sc_segment_sum(zrows, edge_attr, recv)
    p0 = partials[0, :N]
    p1 = partials[1, :N]
    x_ = _tc_mlp(x, p0, p1, W[:D_FEAT], W[D_FEAT:], b[None, :])
    return (x_, edge_attr, edge_index, pos)


# speedup vs baseline: 5.0420x; 1.0937x over previous
import functools

import jax
import jax.numpy as jnp
from jax import lax
from jax.experimental import pallas as pl
from jax.experimental.pallas import tpu as pltpu
from jax.experimental.pallas import tpu_sc as plsc

N = 10000
E = 320000
D_FEAT = 128
D_EDGE = 16

NC = 2
NS = 16
NW = NC * NS
CHUNK = 128
NCHUNKS = E // CHUNK
CPT = NCHUNKS // NW
XTRA = NCHUNKS - CPT * NW
BPB = 6
NBLK = CPT // BPB
BLK = BPB * CHUNK
N_PAD = 10112
ROWS_PER_SUB = N_PAD // NS

_sc_mesh = plsc.VectorSubcoreMesh(core_axis_name="c", subcore_axis_name="s")


@functools.partial(
    pl.kernel,
    out_type=jax.ShapeDtypeStruct((NC, N_PAD, D_EDGE), jnp.float32),
    mesh=_sc_mesh,
    compiler_params=pltpu.CompilerParams(use_tc_tiling_on_sc=False),
    scratch_types=[
        pltpu.VMEM_SHARED((N_PAD, D_EDGE), jnp.float32),
        pltpu.VMEM((BLK, D_EDGE), jnp.float32),
        pltpu.VMEM((BLK, D_EDGE), jnp.float32),
        pltpu.VMEM((BLK,), jnp.int32),
        pltpu.VMEM((BLK,), jnp.int32),
        pltpu.SemaphoreType.DMA,
        pltpu.SemaphoreType.DMA,
    ],
)
def _sc_segment_sum(z_hbm, ea_hbm, recv_hbm, out_hbm, acc, rows_a, rows_b,
                    idx_a, idx_b, sem_a, sem_b):
    c = lax.axis_index("c")
    s = lax.axis_index("s")
    wid = s * NC + c
    base = wid * CPT * CHUNK
    row0 = s * ROWS_PER_SUB

    row_bufs = (rows_a, rows_b)
    idx_bufs = (idx_a, idx_b)
    sems = (sem_a, sem_b)

    def start_load(blk, buf):
        off = base + blk * BLK
        pltpu.async_copy(recv_hbm.at[pl.ds(off, BLK)], idx_bufs[buf],
                         sems[buf])
        pltpu.async_copy(ea_hbm.at[pl.ds(off, BLK)], row_bufs[buf],
                         sems[buf])

    def wait_load(buf):
        pltpu.make_async_copy(recv_hbm.at[pl.ds(0, BLK)], idx_bufs[buf],
                              sems[buf]).wait()
        pltpu.make_async_copy(ea_hbm.at[pl.ds(0, BLK)], row_bufs[buf],
                              sems[buf]).wait()

    def scatter_block(buf):
        for j in range(BPB):
            s0 = j * CHUNK
            pltpu.sync_copy(row_bufs[buf].at[pl.ds(s0, CHUNK)],
                            acc.at[idx_bufs[buf].at[pl.ds(s0, CHUNK)]],
                            add=True)

    start_load(0, 0)
    pltpu.sync_copy(z_hbm.at[pl.ds(row0, ROWS_PER_SUB)],
                    acc.at[pl.ds(row0, ROWS_PER_SUB)])
    plsc.subcore_barrier()

    def _pair(i, _):
        g = 2 * i
        start_load(g + 1, 1)
        wait_load(0)
        scatter_block(0)
        start_load(jnp.minimum(g + 2, NBLK - 1), 0)
        wait_load(1)
        scatter_block(1)
        return 0

    lax.fori_loop(0, NBLK // 2, _pair, 0)
    wait_load(0)
    scatter_block(0)

    def _extra(_, __):
        off = (NW * CPT + wid) * CHUNK
        pltpu.sync_copy(recv_hbm.at[pl.ds(off, CHUNK)],
                        idx_a.at[pl.ds(0, CHUNK)])
        pltpu.sync_copy(ea_hbm.at[pl.ds(off, CHUNK)],
                        rows_a.at[pl.ds(0, CHUNK)])
        pltpu.sync_copy(rows_a.at[pl.ds(0, CHUNK)],
                        acc.at[idx_a.at[pl.ds(0, CHUNK)]], add=True)
        return 0

    lax.fori_loop(0, jnp.where(wid < XTRA, 1, 0), _extra, 0)

    plsc.subcore_barrier()

    pltpu.sync_copy(acc.at[pl.ds(row0, ROWS_PER_SUB)],
                    out_hbm.at[c, pl.ds(row0, ROWS_PER_SUB)])


def _mlp_body(x_ref, p0_ref, p1_ref, wx_ref, we_ref, b_ref, o_ref):
    agg = p0_ref[...] + p1_ref[...]
    o_ref[...] = (
        jnp.dot(x_ref[...], wx_ref[...], preferred_element_type=jnp.float32)
        + jnp.dot(agg, we_ref[...], preferred_element_type=jnp.float32)
        + b_ref[...]
    )


@jax.jit
def _tc_mlp(x, p0, p1, wx, we, b2d):
    return pl.pallas_call(
        _mlp_body,
        out_shape=jax.ShapeDtypeStruct((N, D_FEAT), jnp.float32),
    )(x, p0, p1, wx, we, b2d)


@jax.jit
def kernel(x, edge_index, edge_attr, pos, W, b):
    recv = edge_index[1]
    zrows = jnp.zeros((N_PAD, D_EDGE), jnp.float32)
    partials = _sc_segment_sum(zrows, edge_attr, recv)
    p0 = partials[0, :N]
    p1 = partials[1, :N]
    x_ = _tc_mlp(x, p0, p1, W[:D_FEAT], W[D_FEAT:], b[None, :])
    return (x_, edge_attr, edge_index, pos)
